# baseline (speedup 1.0000x reference)
import jax
import jax.numpy as jnp
from jax import lax
from jax.experimental import pallas as pl
from jax.experimental.pallas import tpu as pltpu
from jax.experimental.pallas import tpu_sc as plsc

N_TIME = 5
N_SAMPLES = 10
N_SPACE = 10000
E_PER = N_SPACE * 16
N_GRAPHS = N_TIME * N_SAMPLES

NC = 2
NS = 16
NW = NC * NS
L = 16

CH = 8000
N_CHUNKS = E_PER // CH
VSTEPS = N_SPACE // L


def _prep_body(d_ref, src_ref, dst_ref, ws_ref, wn_ref,
               std_ref, e_ref, ws16_ref, wn16_ref):
    std_ref[...] = jax.nn.softplus(d_ref[...])
    e_ref[...] = src_ref[...] | (dst_ref[...] << 16)
    ws16_ref[...] = jnp.broadcast_to(ws_ref[...], (L,))
    wn16_ref[...] = jnp.broadcast_to(wn_ref[...], (L,))


def _sc_body(z_hbm, std_hbm, mean_hbm, edges_hbm, ws_hbm, wn_hbm,
             out_hbm, xv1, agg1, xv2, agg2, xpv, stdv, stdv2, ev0, ev1,
             wsv, wnv, sem0, sem1, sem2):
    wid = lax.axis_index("s") * NC + lax.axis_index("c")
    g1 = wid
    g2 = wid + NW
    has2 = g2 < N_GRAPHS
    g2c = jnp.minimum(g2, N_GRAPHS - 1)

    pltpu.async_copy(edges_hbm.at[pl.ds(0, CH)], ev0, sem0)
    pltpu.async_copy(z_hbm.at[g1], xv1, sem2)
    pltpu.async_copy(z_hbm.at[g2c], xv2, sem2)
    pltpu.async_copy(std_hbm.at[lax.rem(g1, N_TIME)], stdv, sem2)
    pltpu.async_copy(std_hbm.at[lax.rem(g2c, N_TIME)], stdv2, sem2)

    pltpu.sync_copy(ws_hbm, wsv)
    pltpu.sync_copy(wn_hbm, wnv)
    ws = wsv[...]
    wn = wnv[...]

    pltpu.make_async_copy(z_hbm.at[g1], xv1, sem2).wait()
    pltpu.make_async_copy(z_hbm.at[g1], xv2, sem2).wait()
    pltpu.make_async_copy(z_hbm.at[g1], stdv, sem2).wait()
    pltpu.make_async_copy(z_hbm.at[g1], stdv2, sem2).wait()

    @plsc.parallel_loop(0, VSTEPS, unroll=8)
    def _(i):
        sl = pl.ds(i * L, L)
        x1 = xv1[sl] * stdv[sl]
        x2 = xv2[sl] * stdv2[sl]
        xv1[sl] = x1
        xv2[sl] = x2
        b1 = plsc.bitcast(x1, jnp.int32) + jnp.int32(0x8000)
        b2 = plsc.bitcast(x2, jnp.int32) + jnp.int32(0x8000)
        xpv[sl] = lax.bitwise_and(b1, jnp.int32(-65536)) | \
            lax.shift_right_logical(b2, jnp.int32(16))
        agg1[sl] = jnp.zeros((L,), jnp.float32)
        agg2[sl] = jnp.zeros((L,), jnp.float32)

    pltpu.async_copy(mean_hbm.at[lax.div(g1, N_SAMPLES)], stdv, sem2)
    pltpu.async_copy(mean_hbm.at[lax.div(g2c, N_SAMPLES)], stdv2, sem2)

    def do_chunk(ev):
        @plsc.parallel_loop(0, CH // L, unroll=24)
        def _(i):
            p = ev[pl.ds(i * L, L)]
            si = lax.bitwise_and(p, jnp.int32(0xFFFF))
            di = lax.shift_right_logical(p, jnp.int32(16))
            q = plsc.load_gather(xpv, [si])
            v1 = plsc.bitcast(lax.bitwise_and(q, jnp.int32(-65536)),
                              jnp.float32)
            v2 = plsc.bitcast(lax.shift_left(q, jnp.int32(16)), jnp.float32)
            plsc.addupdate_scatter(agg1, [di], v1)
            plsc.addupdate_scatter(agg2, [di], v2)

    @pl.loop(0, N_CHUNKS, step=2)
    def _(c):
        @pl.when(c + 1 < N_CHUNKS)
        def _():
            pltpu.async_copy(edges_hbm.at[pl.ds((c + 1) * CH, CH)], ev1, sem1)
        pltpu.make_async_copy(edges_hbm.at[pl.ds(0, CH)], ev0, sem0).wait()
        do_chunk(ev0)

        @pl.when(c + 2 < N_CHUNKS)
        def _():
            pltpu.async_copy(edges_hbm.at[pl.ds((c + 2) * CH, CH)], ev0, sem0)
        pltpu.make_async_copy(edges_hbm.at[pl.ds(0, CH)], ev1, sem1).wait()
        do_chunk(ev1)

    pltpu.make_async_copy(z_hbm.at[g1], stdv, sem2).wait()
    pltpu.make_async_copy(z_hbm.at[g1], stdv2, sem2).wait()

    @plsc.parallel_loop(0, VSTEPS, unroll=8)
    def _(i):
        sl = pl.ds(i * L, L)
        xv1[sl] = ws * xv1[sl] + wn * agg1[sl] + stdv[sl]
        xv2[sl] = ws * xv2[sl] + wn * agg2[sl] + stdv2[sl]

    pltpu.sync_copy(xv1, out_hbm.at[g1])

    @pl.when(has2)
    def _():
        pltpu.sync_copy(xv2, out_hbm.at[g2])


@jax.jit
def kernel(standard_sample, edge_index, mean_param, diag_param,
           post_diag_param, w_self, w_neighbor):
    del post_diag_param

    z2d = standard_sample.reshape(N_GRAPHS, N_SPACE)
    diag2d = diag_param.reshape(N_TIME, N_SPACE)
    mean2d = mean_param.reshape(N_TIME, N_SPACE)
    src = edge_index[0, :E_PER]
    dst = edge_index[1, :E_PER]

    std2d, edges, ws16, wn16 = pl.pallas_call(
        _prep_body,
        out_shape=(
            jax.ShapeDtypeStruct((N_TIME, N_SPACE), jnp.float32),
            jax.ShapeDtypeStruct((E_PER,), jnp.int32),
            jax.ShapeDtypeStruct((L,), jnp.float32),
            jax.ShapeDtypeStruct((L,), jnp.float32),
        ),
    )(diag2d, src, dst, w_self.astype(jnp.float32),
      w_neighbor.astype(jnp.float32))

    mesh = plsc.VectorSubcoreMesh(
        core_axis_name="c", subcore_axis_name="s", num_cores=NC,
        num_subcores=NS)
    sc_call = pl.kernel(
        _sc_body,
        out_type=jax.ShapeDtypeStruct((N_GRAPHS, N_SPACE), jnp.float32),
        mesh=mesh,
        compiler_params=pltpu.CompilerParams(needs_layout_passes=False),
        scratch_types=[
            pltpu.VMEM((N_SPACE,), jnp.float32),
            pltpu.VMEM((N_SPACE,), jnp.float32),
            pltpu.VMEM((N_SPACE,), jnp.float32),
            pltpu.VMEM((N_SPACE,), jnp.float32),
            pltpu.VMEM((N_SPACE,), jnp.int32),
            pltpu.VMEM((N_SPACE,), jnp.float32),
            pltpu.VMEM((N_SPACE,), jnp.float32),
            pltpu.VMEM((CH,), jnp.int32),
            pltpu.VMEM((CH,), jnp.int32),
            pltpu.VMEM((L,), jnp.float32),
            pltpu.VMEM((L,), jnp.float32),
            pltpu.SemaphoreType.DMA,
            pltpu.SemaphoreType.DMA,
            pltpu.SemaphoreType.DMA,
        ],
    )
    out2d = sc_call(z2d, std2d, mean2d, edges, ws16, wn16)
    return out2d.reshape(N_TIME, N_SAMPLES, N_SPACE)


# speedup vs baseline: 1.0344x; 1.0344x over previous
import jax
import jax.numpy as jnp
from jax import lax
from jax.experimental import pallas as pl
from jax.experimental.pallas import tpu as pltpu
from jax.experimental.pallas import tpu_sc as plsc

N_TIME = 5
N_SAMPLES = 10
N_SPACE = 10000
E_PER = N_SPACE * 16
N_GRAPHS = N_TIME * N_SAMPLES

NC = 2
NS = 16
NW = NC * NS
L = 16

CH = 10000
N_CHUNKS = E_PER // CH
VSTEPS = N_SPACE // L


def _prep_body(d_ref, src_ref, dst_ref, ws_ref, wn_ref,
               std_ref, e_ref, ws16_ref, wn16_ref):
    std_ref[...] = jax.nn.softplus(d_ref[...])
    e_ref[...] = src_ref[...] | (dst_ref[...] << 16)
    ws16_ref[...] = jnp.broadcast_to(ws_ref[...], (L,))
    wn16_ref[...] = jnp.broadcast_to(wn_ref[...], (L,))


def _sc_body(z_hbm, std_hbm, mean_hbm, edges_hbm, ws_hbm, wn_hbm,
             out_hbm, xv1, agg1, xv2, agg2, xpv, stdv, stdv2, ev0, ev1,
             wsv, wnv, sem0, sem1, sem2):
    wid = lax.axis_index("s") * NC + lax.axis_index("c")
    g1 = wid
    g2 = wid + NW
    has2 = g2 < N_GRAPHS
    g2c = jnp.minimum(g2, N_GRAPHS - 1)

    pltpu.async_copy(edges_hbm.at[pl.ds(0, CH)], ev0, sem0)
    pltpu.async_copy(z_hbm.at[g1], xv1, sem2)
    pltpu.async_copy(z_hbm.at[g2c], xv2, sem2)
    pltpu.async_copy(std_hbm.at[lax.rem(g1, N_TIME)], stdv, sem2)
    pltpu.async_copy(std_hbm.at[lax.rem(g2c, N_TIME)], stdv2, sem2)

    pltpu.sync_copy(ws_hbm, wsv)
    pltpu.sync_copy(wn_hbm, wnv)
    ws = wsv[...]
    wn = wnv[...]

    pltpu.make_async_copy(z_hbm.at[g1], xv1, sem2).wait()
    pltpu.make_async_copy(z_hbm.at[g1], xv2, sem2).wait()
    pltpu.make_async_copy(z_hbm.at[g1], stdv, sem2).wait()
    pltpu.make_async_copy(z_hbm.at[g1], stdv2, sem2).wait()

    @plsc.parallel_loop(0, VSTEPS, unroll=8)
    def _(i):
        sl = pl.ds(i * L, L)
        x1 = xv1[sl] * stdv[sl]
        x2 = xv2[sl] * stdv2[sl]
        xv1[sl] = x1
        xv2[sl] = x2
        b1 = plsc.bitcast(x1, jnp.int32) + jnp.int32(0x8000)
        b2 = plsc.bitcast(x2, jnp.int32) + jnp.int32(0x8000)
        xpv[sl] = lax.bitwise_and(b1, jnp.int32(-65536)) | \
            lax.shift_right_logical(b2, jnp.int32(16))
        agg1[sl] = jnp.zeros((L,), jnp.float32)
        agg2[sl] = jnp.zeros((L,), jnp.float32)

    pltpu.async_copy(mean_hbm.at[lax.div(g1, N_SAMPLES)], stdv, sem2)
    pltpu.async_copy(mean_hbm.at[lax.div(g2c, N_SAMPLES)], stdv2, sem2)

    def do_chunk(ev):
        @plsc.parallel_loop(0, CH // L, unroll=16)
        def _(i):
            p = ev[pl.ds(i * L, L)]
            si = lax.bitwise_and(p, jnp.int32(0xFFFF))
            di = lax.shift_right_logical(p, jnp.int32(16))
            q = plsc.load_gather(xpv, [si])
            v1 = plsc.bitcast(lax.bitwise_and(q, jnp.int32(-65536)),
                              jnp.float32)
            v2 = plsc.bitcast(lax.shift_left(q, jnp.int32(16)), jnp.float32)
            plsc.addupdate_scatter(agg1, [di], v1)
            plsc.addupdate_scatter(agg2, [di], v2)

    @pl.loop(0, N_CHUNKS, step=2)
    def _(c):
        @pl.when(c + 1 < N_CHUNKS)
        def _():
            pltpu.async_copy(edges_hbm.at[pl.ds((c + 1) * CH, CH)], ev1, sem1)
        pltpu.make_async_copy(edges_hbm.at[pl.ds(0, CH)], ev0, sem0).wait()
        do_chunk(ev0)

        @pl.when(c + 2 < N_CHUNKS)
        def _():
            pltpu.async_copy(edges_hbm.at[pl.ds((c + 2) * CH, CH)], ev0, sem0)
        pltpu.make_async_copy(edges_hbm.at[pl.ds(0, CH)], ev1, sem1).wait()
        do_chunk(ev1)

    pltpu.make_async_copy(z_hbm.at[g1], stdv, sem2).wait()
    pltpu.make_async_copy(z_hbm.at[g1], stdv2, sem2).wait()

    @plsc.parallel_loop(0, VSTEPS, unroll=8)
    def _(i):
        sl = pl.ds(i * L, L)
        xv1[sl] = ws * xv1[sl] + wn * agg1[sl] + stdv[sl]
        xv2[sl] = ws * xv2[sl] + wn * agg2[sl] + stdv2[sl]

    pltpu.sync_copy(xv1, out_hbm.at[g1])

    @pl.when(has2)
    def _():
        pltpu.sync_copy(xv2, out_hbm.at[g2])


@jax.jit
def kernel(standard_sample, edge_index, mean_param, diag_param,
           post_diag_param, w_self, w_neighbor):
    del post_diag_param

    z2d = standard_sample.reshape(N_GRAPHS, N_SPACE)
    diag2d = diag_param.reshape(N_TIME, N_SPACE)
    mean2d = mean_param.reshape(N_TIME, N_SPACE)
    src = edge_index[0, :E_PER]
    dst = edge_index[1, :E_PER]

    std2d, edges, ws16, wn16 = pl.pallas_call(
        _prep_body,
        out_shape=(
            jax.ShapeDtypeStruct((N_TIME, N_SPACE), jnp.float32),
            jax.ShapeDtypeStruct((E_PER,), jnp.int32),
            jax.ShapeDtypeStruct((L,), jnp.float32),
            jax.ShapeDtypeStruct((L,), jnp.float32),
        ),
    )(diag2d, src, dst, w_self.astype(jnp.float32),
      w_neighbor.astype(jnp.float32))

    mesh = plsc.VectorSubcoreMesh(
        core_axis_name="c", subcore_axis_name="s", num_cores=NC,
        num_subcores=NS)
    sc_call = pl.kernel(
        _sc_body,
        out_type=jax.ShapeDtypeStruct((N_GRAPHS, N_SPACE), jnp.float32),
        mesh=mesh,
        compiler_params=pltpu.CompilerParams(needs_layout_passes=False),
        scratch_types=[
            pltpu.VMEM((N_SPACE,), jnp.float32),
            pltpu.VMEM((N_SPACE,), jnp.float32),
            pltpu.VMEM((N_SPACE,), jnp.float32),
            pltpu.VMEM((N_SPACE,), jnp.float32),
            pltpu.VMEM((N_SPACE,), jnp.int32),
            pltpu.VMEM((N_SPACE,), jnp.float32),
            pltpu.VMEM((N_SPACE,), jnp.float32),
            pltpu.VMEM((CH,), jnp.int32),
            pltpu.VMEM((CH,), jnp.int32),
            pltpu.VMEM((L,), jnp.float32),
            pltpu.VMEM((L,), jnp.float32),
            pltpu.SemaphoreType.DMA,
            pltpu.SemaphoreType.DMA,
            pltpu.SemaphoreType.DMA,
        ],
    )
    out2d = sc_call(z2d, std2d, mean2d, edges, ws16, wn16)
    return out2d.reshape(N_TIME, N_SAMPLES, N_SPACE)


# speedup vs baseline: 1.0357x; 1.0013x over previous
import jax
import jax.numpy as jnp
from jax import lax
from jax.experimental import pallas as pl
from jax.experimental.pallas import tpu as pltpu
from jax.experimental.pallas import tpu_sc as plsc

N_TIME = 5
N_SAMPLES = 10
N_SPACE = 10000
E_PER = N_SPACE * 16
N_GRAPHS = N_TIME * N_SAMPLES

NC = 2
NS = 16
NW = NC * NS
L = 16

CH = 10000
N_CHUNKS = E_PER // CH
VSTEPS = N_SPACE // L


def _prep_body(d_ref, src_ref, dst_ref, ws_ref, wn_ref,
               std_ref, e_ref, ws16_ref, wn16_ref):
    std_ref[...] = jax.nn.softplus(d_ref[...])
    e_ref[...] = src_ref[...] | (dst_ref[...] << 16)
    ws16_ref[...] = jnp.broadcast_to(ws_ref[...], (L,))
    wn16_ref[...] = jnp.broadcast_to(wn_ref[...], (L,))


def _sc_body(z_hbm, std_hbm, mean_hbm, edges_hbm, ws_hbm, wn_hbm,
             out_hbm, xv1, agg1, xv2, agg2, xpv, stdv, stdv2, ev0, ev1,
             wsv, wnv, sem0, sem1, sem2):
    wid = lax.axis_index("s") * NC + lax.axis_index("c")
    g1 = wid
    g2 = wid + NW
    has2 = g2 < N_GRAPHS
    g2c = jnp.minimum(g2, N_GRAPHS - 1)

    pltpu.async_copy(edges_hbm.at[pl.ds(0, CH)], ev0, sem0)
    pltpu.async_copy(z_hbm.at[g1], xv1, sem2)
    pltpu.async_copy(z_hbm.at[g2c], xv2, sem2)
    pltpu.async_copy(std_hbm.at[lax.rem(g1, N_TIME)], stdv, sem2)
    pltpu.async_copy(std_hbm.at[lax.rem(g2c, N_TIME)], stdv2, sem2)

    pltpu.sync_copy(ws_hbm, wsv)
    pltpu.sync_copy(wn_hbm, wnv)
    ws = wsv[...]
    wn = wnv[...]

    pltpu.make_async_copy(z_hbm.at[g1], xv1, sem2).wait()
    pltpu.make_async_copy(z_hbm.at[g1], xv2, sem2).wait()
    pltpu.make_async_copy(z_hbm.at[g1], stdv, sem2).wait()
    pltpu.make_async_copy(z_hbm.at[g1], stdv2, sem2).wait()

    @plsc.parallel_loop(0, VSTEPS, unroll=8)
    def _(i):
        sl = pl.ds(i * L, L)
        x1 = xv1[sl] * stdv[sl]
        x2 = xv2[sl] * stdv2[sl]
        xv1[sl] = x1
        xv2[sl] = x2
        b1 = plsc.bitcast(x1, jnp.int32) + jnp.int32(0x8000)
        b2 = plsc.bitcast(x2, jnp.int32) + jnp.int32(0x8000)
        xpv[sl] = lax.bitwise_and(b1, jnp.int32(-65536)) | \
            lax.shift_right_logical(b2, jnp.int32(16))
        agg1[sl] = jnp.zeros((L,), jnp.float32)
        agg2[sl] = jnp.zeros((L,), jnp.float32)

    pltpu.async_copy(mean_hbm.at[lax.div(g1, N_SAMPLES)], stdv, sem2)
    pltpu.async_copy(mean_hbm.at[lax.div(g2c, N_SAMPLES)], stdv2, sem2)

    def do_chunk(ev):
        @plsc.parallel_loop(0, CH // L, unroll=16)
        def _(i):
            p = ev[pl.ds(i * L, L)]
            si = lax.bitwise_and(p, jnp.int32(0xFFFF))
            di = lax.shift_right_logical(p, jnp.int32(16))
            q = plsc.load_gather(xpv, [si])
            v1 = plsc.bitcast(lax.bitwise_and(q, jnp.int32(-65536)),
                              jnp.float32)
            v2 = plsc.bitcast(lax.shift_left(q, jnp.int32(16)), jnp.float32)
            plsc.addupdate_scatter(agg1, [di], v1)
            plsc.addupdate_scatter(agg2, [di], v2)

    @pl.loop(0, N_CHUNKS, step=2)
    def _(c):
        @pl.when(c + 1 < N_CHUNKS)
        def _():
            pltpu.async_copy(edges_hbm.at[pl.ds((c + 1) * CH, CH)], ev1, sem1)
        pltpu.make_async_copy(edges_hbm.at[pl.ds(0, CH)], ev0, sem0).wait()
        do_chunk(ev0)

        @pl.when(c + 2 < N_CHUNKS)
        def _():
            pltpu.async_copy(edges_hbm.at[pl.ds((c + 2) * CH, CH)], ev0, sem0)
        pltpu.make_async_copy(edges_hbm.at[pl.ds(0, CH)], ev1, sem1).wait()
        do_chunk(ev1)

    pltpu.make_async_copy(z_hbm.at[g1], stdv, sem2).wait()
    pltpu.make_async_copy(z_hbm.at[g1], stdv2, sem2).wait()

    @plsc.parallel_loop(0, VSTEPS, unroll=8)
    def _(i):
        sl = pl.ds(i * L, L)
        xv1[sl] = ws * xv1[sl] + wn * agg1[sl] + stdv[sl]
        xv2[sl] = ws * xv2[sl] + wn * agg2[sl] + stdv2[sl]

    pltpu.sync_copy(xv1, out_hbm.at[g1])

    @pl.when(has2)
    def _():
        pltpu.sync_copy(xv2, out_hbm.at[g2])


@jax.jit
def kernel(standard_sample, edge_index, mean_param, diag_param,
           post_diag_param, w_self, w_neighbor):
    del post_diag_param

    z2d = standard_sample.reshape(N_GRAPHS, N_SPACE)
    diag2d = diag_param.reshape(N_TIME, N_SPACE)
    mean2d = mean_param.reshape(N_TIME, N_SPACE)
    src = edge_index[0, :E_PER]
    dst = edge_index[1, :E_PER]

    std2d, edges, ws16, wn16 = pl.pallas_call(
        _prep_body,
        out_shape=(
            jax.ShapeDtypeStruct((N_TIME, N_SPACE), jnp.float32),
            jax.ShapeDtypeStruct((E_PER,), jnp.int32),
            jax.ShapeDtypeStruct((L,), jnp.float32),
            jax.ShapeDtypeStruct((L,), jnp.float32),
        ),
    )(diag2d, src, dst, w_self.astype(jnp.float32),
      w_neighbor.astype(jnp.float32))

    mesh = plsc.VectorSubcoreMesh(
        core_axis_name="c", subcore_axis_name="s", num_cores=NC,
        num_subcores=NS)
    sc_call = pl.kernel(
        _sc_body,
        out_type=jax.ShapeDtypeStruct((N_GRAPHS, N_SPACE), jnp.float32),
        mesh=mesh,
        compiler_params=pltpu.CompilerParams(needs_layout_passes=False, disable_bounds_checks=True, disable_semaphore_checks=True),
        scratch_types=[
            pltpu.VMEM((N_SPACE,), jnp.float32),
            pltpu.VMEM((N_SPACE,), jnp.float32),
            pltpu.VMEM((N_SPACE,), jnp.float32),
            pltpu.VMEM((N_SPACE,), jnp.float32),
            pltpu.VMEM((N_SPACE,), jnp.int32),
            pltpu.VMEM((N_SPACE,), jnp.float32),
            pltpu.VMEM((N_SPACE,), jnp.float32),
            pltpu.VMEM((CH,), jnp.int32),
            pltpu.VMEM((CH,), jnp.int32),
            pltpu.VMEM((L,), jnp.float32),
            pltpu.VMEM((L,), jnp.float32),
            pltpu.SemaphoreType.DMA,
            pltpu.SemaphoreType.DMA,
            pltpu.SemaphoreType.DMA,
        ],
    )
    out2d = sc_call(z2d, std2d, mean2d, edges, ws16, wn16)
    return out2d.reshape(N_TIME, N_SAMPLES, N_SPACE)
